# baseline (speedup 1.0000x reference)
import functools

import jax
import jax.numpy as jnp
from jax import lax
from jax.experimental import pallas as pl
from jax.experimental.pallas import tpu as pltpu
from jax.experimental.pallas import tpu_sc as plsc

B = 32768
D_TEA = 768
GATE_H = 512
NUM_EXPERTS = 64
TOP_K = 8
BLK = 1024

NW = 32
S = 1
B_SEG = B // S
TOK_PER_W = B_SEG // NW
L = 16
U = 4


def _gate_block(h_ref, w1_ref, b1_ref, w2_ref, b2_ref, pt_ref, pk_ref):
    h = h_ref[...]
    hidden = jnp.dot(h, w1_ref[...], preferred_element_type=jnp.float32)
    hidden = hidden + b1_ref[...]
    hidden = 0.5 * hidden * (1.0 + jax.lax.erf(hidden * (2.0 ** -0.5)))
    logits = jnp.dot(hidden, w2_ref[...], preferred_element_type=jnp.float32)
    logits = logits + b2_ref[...]

    m = jnp.max(logits, axis=-1, keepdims=True)
    e = jnp.exp(logits - m)
    probs = e / jnp.sum(e, axis=-1, keepdims=True)
    pt_ref[...] = probs.T
    pk_ref[...] = jnp.concatenate(
        [probs[:BLK // 2, :], probs[BLK // 2:, :]], axis=1)


def _tc_gate_seg(seg):
    blk0 = seg * (B_SEG // BLK)
    return pl.pallas_call(
        _gate_block,
        grid=(B_SEG // BLK,),
        in_specs=[
            pl.BlockSpec((BLK, D_TEA), lambda i: (i + blk0, 0)),
            pl.BlockSpec((D_TEA, GATE_H), lambda i: (0, 0)),
            pl.BlockSpec((GATE_H,), lambda i: (0,)),
            pl.BlockSpec((GATE_H, NUM_EXPERTS), lambda i: (0, 0)),
            pl.BlockSpec((NUM_EXPERTS,), lambda i: (0,)),
        ],
        out_specs=[
            pl.BlockSpec((NUM_EXPERTS, BLK), lambda i: (0, i)),
            pl.BlockSpec((BLK // 2, 2 * NUM_EXPERTS), lambda i: (i, 0)),
        ],
        out_shape=[
            jax.ShapeDtypeStruct((NUM_EXPERTS, B_SEG), jnp.float32),
            jax.ShapeDtypeStruct((B_SEG // 2, 2 * NUM_EXPERTS), jnp.float32),
        ],
    )


_GDN = lax.GatherDimensionNumbers(
    offset_dims=(), collapsed_slice_dims=(0,), start_index_map=(0,))


def _sc_topk_body(probs_hbm, tkw_hbm, tki_hbm, pv, ow, oi):
    wid = lax.axis_index("s") * 2 + lax.axis_index("c")
    base = wid * TOK_PER_W
    lane = lax.iota(jnp.int32, L)
    lowmask = lane < TOP_K
    perm = (lane + TOP_K) % L

    rev_perm = (7 - lane) & (L - 1)

    def permute(x, p):
        return lax.gather(x, p[:, None], _GDN, (1,),
                          mode=lax.GatherScatterMode.PROMISE_IN_BOUNDS)

    def pair_top8(ka, va, kb, vb):
        rkb = permute(kb, rev_perm)
        rvb = permute(vb, rev_perm)
        sel = ka >= rkb
        return jnp.where(sel, ka, rkb), jnp.where(sel, va, rvb)

    def one_tok(dat_off, out_off):
        ks = []
        vs = []
        for g in range(4):
            kg = pv[pl.ds(dat_off + g * L, L)]
            sk, sv = plsc.sort_key_val(kg, lane + g * L, descending=True)
            ks.append(sk)
            vs.append(sv)
        ak, av = pair_top8(ks[0], vs[0], ks[1], vs[1])
        bk, bv = pair_top8(ks[2], vs[2], ks[3], vs[3])
        ck = jnp.where(lowmask, ak, permute(bk, perm))
        cv = jnp.where(lowmask, av, permute(bv, perm))
        fk, fv = plsc.sort_key_val(ck, cv, descending=True)
        w = jnp.where(lowmask, fk, 0.0)
        tkw = w / (jnp.sum(w) + 1e-08)
        plsc.store_compressed(ow.at[pl.ds(out_off, L)], tkw, mask=lowmask)
        plsc.store_compressed(oi.at[pl.ds(out_off, L)], fv, mask=lowmask)

    pltpu.sync_copy(
        probs_hbm.at[pl.ds(base * NUM_EXPERTS, TOK_PER_W * NUM_EXPERTS)], pv)

    HALF = TOK_PER_W // 2

    @plsc.parallel_loop(0, HALF, step=1, unroll=U)
    def _loop(i):
        one_tok(i * 2 * NUM_EXPERTS, i * TOP_K)
        one_tok(i * 2 * NUM_EXPERTS + NUM_EXPERTS, (HALF + i) * TOP_K)
    pltpu.sync_copy(ow.at[pl.ds(0, TOK_PER_W * TOP_K)],
                    tkw_hbm.at[pl.ds(base * TOP_K, TOK_PER_W * TOP_K)])
    pltpu.sync_copy(oi.at[pl.ds(0, TOK_PER_W * TOP_K)],
                    tki_hbm.at[pl.ds(base * TOP_K, TOK_PER_W * TOP_K)])


_sc_topk = functools.partial(
    pl.kernel,
    out_type=[
        jax.ShapeDtypeStruct((B_SEG * TOP_K,), jnp.float32),
        jax.ShapeDtypeStruct((B_SEG * TOP_K,), jnp.int32),
    ],
    mesh=plsc.VectorSubcoreMesh(core_axis_name="c", subcore_axis_name="s"),
    compiler_params=pltpu.CompilerParams(needs_layout_passes=False),
    scratch_types=[
        pltpu.VMEM((TOK_PER_W * NUM_EXPERTS,), jnp.float32),
        pltpu.VMEM((TOK_PER_W * TOP_K + TOP_K,), jnp.float32),
        pltpu.VMEM((TOK_PER_W * TOP_K + TOP_K,), jnp.int32),
    ],
)(_sc_topk_body)


def _kernel_impl(h_pooled, W1, b1, W2, b2):
    probs_segs = []
    tkw_segs = []
    tki_segs = []
    for s in range(S):
        pt, pk = _tc_gate_seg(s)(h_pooled, W1, b1, W2, b2)
        w, i = _sc_topk(pk.reshape(-1))
        probs_segs.append(pt.T)
        tkw_segs.append(w.reshape(B_SEG, TOP_K))
        tki_segs.append(i.reshape(B_SEG, TOP_K))
    return (jnp.concatenate(tkw_segs, axis=0),
            jnp.concatenate(tki_segs, axis=0),
            jnp.concatenate(probs_segs, axis=0))


kernel = jax.jit(_kernel_impl)


# speedup vs baseline: 1.0126x; 1.0126x over previous
import functools

import jax
import jax.numpy as jnp
from jax import lax
from jax.experimental import pallas as pl
from jax.experimental.pallas import tpu as pltpu
from jax.experimental.pallas import tpu_sc as plsc

B = 32768
D_TEA = 768
GATE_H = 512
NUM_EXPERTS = 64
TOP_K = 8
BLK = 1024

NW = 32
S = 1
B_SEG = B // S
TOK_PER_W = B_SEG // NW
L = 16
U = 2


def _gate_block(h_ref, w1_ref, b1_ref, w2_ref, b2_ref, pt_ref, pk_ref):
    h = h_ref[...]
    hidden = jnp.dot(h, w1_ref[...], preferred_element_type=jnp.float32)
    hidden = hidden + b1_ref[...]
    hidden = 0.5 * hidden * (1.0 + jax.lax.erf(hidden * (2.0 ** -0.5)))
    logits = jnp.dot(hidden, w2_ref[...], preferred_element_type=jnp.float32)
    logits = logits + b2_ref[...]

    m = jnp.max(logits, axis=-1, keepdims=True)
    e = jnp.exp(logits - m)
    probs = e / jnp.sum(e, axis=-1, keepdims=True)
    pt_ref[...] = probs.T
    pk_ref[...] = jnp.concatenate(
        [probs[:BLK // 2, :], probs[BLK // 2:, :]], axis=1)


def _tc_gate_seg(seg):
    blk0 = seg * (B_SEG // BLK)
    return pl.pallas_call(
        _gate_block,
        grid=(B_SEG // BLK,),
        in_specs=[
            pl.BlockSpec((BLK, D_TEA), lambda i: (i + blk0, 0)),
            pl.BlockSpec((D_TEA, GATE_H), lambda i: (0, 0)),
            pl.BlockSpec((GATE_H,), lambda i: (0,)),
            pl.BlockSpec((GATE_H, NUM_EXPERTS), lambda i: (0, 0)),
            pl.BlockSpec((NUM_EXPERTS,), lambda i: (0,)),
        ],
        out_specs=[
            pl.BlockSpec((NUM_EXPERTS, BLK), lambda i: (0, i)),
            pl.BlockSpec((BLK // 2, 2 * NUM_EXPERTS), lambda i: (i, 0)),
        ],
        out_shape=[
            jax.ShapeDtypeStruct((NUM_EXPERTS, B_SEG), jnp.float32),
            jax.ShapeDtypeStruct((B_SEG // 2, 2 * NUM_EXPERTS), jnp.float32),
        ],
    )


_GDN = lax.GatherDimensionNumbers(
    offset_dims=(), collapsed_slice_dims=(0,), start_index_map=(0,))


def _sc_topk_body(probs_hbm, tkw_hbm, tki_hbm, pv, ow, oi):
    wid = lax.axis_index("s") * 2 + lax.axis_index("c")
    base = wid * TOK_PER_W
    lane = lax.iota(jnp.int32, L)
    lowmask = lane < TOP_K
    perm = (lane + TOP_K) % L

    rev_perm = (7 - lane) & (L - 1)

    def permute(x, p):
        return lax.gather(x, p[:, None], _GDN, (1,),
                          mode=lax.GatherScatterMode.PROMISE_IN_BOUNDS)

    def pair_top8(ka, va, kb, vb):
        rkb = permute(kb, rev_perm)
        rvb = permute(vb, rev_perm)
        sel = ka >= rkb
        return jnp.where(sel, ka, rkb), jnp.where(sel, va, rvb)

    def one_tok(dat_off, out_off):
        ks = []
        vs = []
        for g in range(4):
            kg = pv[pl.ds(dat_off + g * L, L)]
            sk, sv = plsc.sort_key_val(kg, lane + g * L, descending=True)
            ks.append(sk)
            vs.append(sv)
        ak, av = pair_top8(ks[0], vs[0], ks[1], vs[1])
        bk, bv = pair_top8(ks[2], vs[2], ks[3], vs[3])
        ck = jnp.where(lowmask, ak, permute(bk, perm))
        cv = jnp.where(lowmask, av, permute(bv, perm))
        fk, fv = plsc.sort_key_val(ck, cv, descending=True)
        w = jnp.where(lowmask, fk, 0.0)
        tkw = w / (jnp.sum(w) + 1e-08)
        plsc.store_compressed(ow.at[pl.ds(out_off, L)], tkw, mask=lowmask)
        plsc.store_compressed(oi.at[pl.ds(out_off, L)], fv, mask=lowmask)

    pltpu.sync_copy(
        probs_hbm.at[pl.ds(base * NUM_EXPERTS, TOK_PER_W * NUM_EXPERTS)], pv)

    HALF = TOK_PER_W // 2

    @plsc.parallel_loop(0, HALF, step=1, unroll=U)
    def _loop(i):
        one_tok(i * 2 * NUM_EXPERTS, i * TOP_K)
        one_tok(i * 2 * NUM_EXPERTS + NUM_EXPERTS, (HALF + i) * TOP_K)
    pltpu.sync_copy(ow.at[pl.ds(0, TOK_PER_W * TOP_K)],
                    tkw_hbm.at[pl.ds(base * TOP_K, TOK_PER_W * TOP_K)])
    pltpu.sync_copy(oi.at[pl.ds(0, TOK_PER_W * TOP_K)],
                    tki_hbm.at[pl.ds(base * TOP_K, TOK_PER_W * TOP_K)])


_sc_topk = functools.partial(
    pl.kernel,
    out_type=[
        jax.ShapeDtypeStruct((B_SEG * TOP_K,), jnp.float32),
        jax.ShapeDtypeStruct((B_SEG * TOP_K,), jnp.int32),
    ],
    mesh=plsc.VectorSubcoreMesh(core_axis_name="c", subcore_axis_name="s"),
    compiler_params=pltpu.CompilerParams(needs_layout_passes=False),
    scratch_types=[
        pltpu.VMEM((TOK_PER_W * NUM_EXPERTS,), jnp.float32),
        pltpu.VMEM((TOK_PER_W * TOP_K + TOP_K,), jnp.float32),
        pltpu.VMEM((TOK_PER_W * TOP_K + TOP_K,), jnp.int32),
    ],
)(_sc_topk_body)


def _kernel_impl(h_pooled, W1, b1, W2, b2):
    probs_segs = []
    tkw_segs = []
    tki_segs = []
    for s in range(S):
        pt, pk = _tc_gate_seg(s)(h_pooled, W1, b1, W2, b2)
        w, i = _sc_topk(pk.reshape(-1))
        probs_segs.append(pt.T)
        tkw_segs.append(w.reshape(B_SEG, TOP_K))
        tki_segs.append(i.reshape(B_SEG, TOP_K))
    return (jnp.concatenate(tkw_segs, axis=0),
            jnp.concatenate(tki_segs, axis=0),
            jnp.concatenate(probs_segs, axis=0))


kernel = jax.jit(_kernel_impl)
